# baseline (speedup 1.0000x reference)
import functools

import jax
import jax.numpy as jnp
from jax import lax
from jax.experimental import pallas as pl
from jax.experimental.pallas import tpu as pltpu
from jax.experimental.pallas import tpu_sc as plsc

N = 100000
E = 1600000
F_IN = 128
HID = 16
K = 3
T = 4

NC, NS, L = 2, 16, 16
NW = NC * NS
N_PAD = 100352
NPT = N_PAD // NS
E_PAD = 1605632
E2 = E_PAD // 128
EPT = E_PAD // NW
MROWS = 56
MEDG = MROWS * 128
NMAC = EPT // MEDG
F32 = jnp.float32
I32 = jnp.int32

_sc_mesh = plsc.VectorSubcoreMesh(core_axis_name="c", subcore_axis_name="s")


def _wid():
    return lax.axis_index("s") * NC + lax.axis_index("c")


def _bcast_lane(nv, e):
    return lax.gather(
        nv, jnp.full((L, 1), e, I32),
        lax.GatherDimensionNumbers(
            offset_dims=(), collapsed_slice_dims=(0,), start_index_map=(0,)),
        (1,), mode=lax.GatherScatterMode.PROMISE_IN_BOUNDS)


@functools.partial(
    pl.kernel,
    out_type=jax.ShapeDtypeStruct((NC, N_PAD), F32),
    mesh=_sc_mesh,
    compiler_params=pltpu.CompilerParams(use_tc_tiling_on_sc=False),
    scratch_types=[
        pltpu.VMEM_SHARED((N_PAD,), F32),
        pltpu.VMEM((MROWS, 128), I32),
        pltpu.VMEM((MROWS, 128), F32),
        pltpu.VMEM((NPT,), F32),
        pltpu.SemaphoreType.DMA,
    ],
)
def _s1_deg(dst2d, ea2d, deg_out, sp_deg, idxb, valb, zbuf, sem):
    cid = lax.axis_index("c")
    sid = lax.axis_index("s")
    wid = _wid()

    def zb(i, _):
        zbuf[pl.ds(i * L, L)] = jnp.zeros((L,), F32)
        return 0
    lax.fori_loop(0, NPT // L, zb, 0)
    pltpu.sync_copy(zbuf, sp_deg.at[pl.ds(sid * NPT, NPT)])
    plsc.subcore_barrier()

    row0 = wid * (EPT // 128)

    def macro(m, _):
        base = row0 + m * MROWS
        d1 = pltpu.async_copy(dst2d.at[pl.ds(base, MROWS)], idxb, sem)
        d2 = pltpu.async_copy(ea2d.at[pl.ds(base, MROWS)], valb, sem)
        d1.wait()
        d2.wait()
        descs = []
        for j in range(MROWS):
            descs.append(pltpu.async_copy(
                valb.at[j], sp_deg.at[idxb.at[j]], sem, add=True))
        for d in descs:
            d.wait()
        return 0
    lax.fori_loop(0, NMAC, macro, 0)
    plsc.subcore_barrier()
    pltpu.sync_copy(sp_deg.at[pl.ds(sid * NPT, NPT)],
                    deg_out.at[cid, pl.ds(sid * NPT, NPT)])


@functools.partial(
    pl.kernel,
    out_type=jax.ShapeDtypeStruct((E2, 128), F32),
    mesh=_sc_mesh,
    compiler_params=pltpu.CompilerParams(use_tc_tiling_on_sc=False),
    scratch_types=[
        pltpu.VMEM_SHARED((N_PAD,), F32),
        pltpu.VMEM((MROWS, 128), I32),
        pltpu.VMEM((MROWS, 128), I32),
        pltpu.VMEM((MROWS, 128), F32),
        pltpu.VMEM((MROWS, 128), F32),
        pltpu.VMEM((MROWS, 128), F32),
        pltpu.SemaphoreType.DMA,
    ],
)
def _s2_norm(dis_tab, src2d, dst2d, ea2d, norm_out,
             sp_dis, sidx, didx, eab, gsb, gdb, sem):
    sid = lax.axis_index("s")
    wid = _wid()

    pltpu.sync_copy(dis_tab.at[pl.ds(sid * NPT, NPT)],
                    sp_dis.at[pl.ds(sid * NPT, NPT)])
    plsc.subcore_barrier()

    row0 = wid * (EPT // 128)

    def macro(m, _):
        base = row0 + m * MROWS
        c1 = pltpu.async_copy(src2d.at[pl.ds(base, MROWS)], sidx, sem)
        c2 = pltpu.async_copy(dst2d.at[pl.ds(base, MROWS)], didx, sem)
        c3 = pltpu.async_copy(ea2d.at[pl.ds(base, MROWS)], eab, sem)
        c1.wait()
        c2.wait()
        c3.wait()
        descs = []
        for j in range(MROWS):
            descs.append(pltpu.async_copy(sp_dis.at[sidx.at[j]], gsb.at[j], sem))
            descs.append(pltpu.async_copy(sp_dis.at[didx.at[j]], gdb.at[j], sem))
        for d in descs:
            d.wait()

        def mul(r, _):
            for c in range(8):
                s = pl.ds(c * L, L)
                eab[r, s] = gsb[r, s] * eab[r, s] * gdb[r, s]
            return 0
        lax.fori_loop(0, MROWS, mul, 0)
        pltpu.sync_copy(eab, norm_out.at[pl.ds(base, MROWS)])
        return 0
    lax.fori_loop(0, NMAC, macro, 0)


MR3 = 4
MEDG3 = MR3 * 128
NM3 = EPT // MEDG3


@functools.partial(
    pl.kernel,
    out_type=jax.ShapeDtypeStruct((NC, K, N_PAD, HID), F32),
    mesh=_sc_mesh,
    compiler_params=pltpu.CompilerParams(use_tc_tiling_on_sc=False),
    scratch_types=[
        pltpu.VMEM_SHARED((N_PAD, HID), F32),
        pltpu.VMEM((2, MR3, 128), I32),
        pltpu.VMEM((2, MR3, 128), I32),
        pltpu.VMEM((3, MR3, 128), I32),
        pltpu.VMEM((2, MR3, 128), F32),
        pltpu.VMEM((3, MEDG3, HID), F32),
        pltpu.SemaphoreType.DMA,
        pltpu.SemaphoreType.DMA,
        pltpu.SemaphoreType.DMA,
        pltpu.SemaphoreType.DMA,
        pltpu.SemaphoreType.DMA,
    ],
)
def _s3_msg(h_tab, src2d, dst2d, norm2d, parts,
            sp_agg, sidx, gidx, didx, nrm, rows, semL, semD, semG, semS,
            semZ):
    cid = lax.axis_index("c")
    sid = lax.axis_index("s")
    wid = _wid()
    row0 = wid * (EPT // 128)

    def fire_l(m, lp):
        base = jnp.minimum(row0 + m * MR3, jnp.int32(E2 - MR3))
        pltpu.async_copy(src2d.at[pl.ds(base, MR3)], sidx.at[lp], semL)
        pltpu.async_copy(norm2d.at[pl.ds(base, MR3)], nrm.at[lp], semL)

    def wait_l():
        for _ in range(2):
            pltpu.make_async_copy(
                src2d.at[pl.ds(0, MR3)], sidx.at[0], semL).wait()

    def fire_d(m, dp):
        base = jnp.minimum(row0 + m * MR3, jnp.int32(E2 - MR3))
        pltpu.async_copy(dst2d.at[pl.ds(base, MR3)], didx.at[dp], semD)

    def wait_d():
        pltpu.make_async_copy(
            dst2d.at[pl.ds(0, MR3)], didx.at[0], semD).wait()

    def addk(lp, koff):
        def body(r, _):
            for c in range(8):
                s = pl.ds(c * L, L)
                gidx[lp, r, s] = sidx[lp, r, s] + koff
            return 0
        lax.fori_loop(0, MR3, body, 0)

    def fire_g(lp, rp):
        for j in range(MR3):
            pltpu.async_copy(h_tab.at[gidx.at[lp].at[j]],
                             rows.at[rp].at[pl.ds(j * 128, 128)], semG)

    def drain_g(rp):
        for j in range(MR3):
            pltpu.make_async_copy(h_tab.at[gidx.at[0].at[j]],
                                  rows.at[rp].at[pl.ds(j * 128, 128)],
                                  semG).wait()

    def scale(rp, lp):
        def body(g, _):
            nv = nrm[lp, g // 8, pl.ds((g % 8) * L, L)]
            for e in range(L):
                i = g * L + e
                rows[rp, i, :] = rows[rp, i, :] * _bcast_lane(nv, e)
            return 0
        lax.fori_loop(0, MEDG3 // L, body, 0)

    def fire_s(rp, dp):
        for j in range(MR3):
            pltpu.async_copy(rows.at[rp].at[pl.ds(j * 128, 128)],
                             sp_agg.at[didx.at[dp].at[j]], semS, add=True)

    def drain_s():
        for j in range(MR3):
            pltpu.make_async_copy(rows.at[0].at[pl.ds(j * 128, 128)],
                                  sp_agg.at[didx.at[0].at[j]], semS).wait()

    for k in range(K):
        koff = jnp.int32(k * N_PAD)
        def zb(i, _):
            rows[0, i, :] = jnp.zeros((L,), F32)
            return 0
        lax.fori_loop(0, MEDG3, zb, 0)
        for z in range(12):
            pltpu.async_copy(
                rows.at[0],
                sp_agg.at[pl.ds(sid * NPT + z * MEDG3, MEDG3)], semZ)
        pltpu.async_copy(
            rows.at[0].at[pl.ds(0, 128)],
            sp_agg.at[pl.ds(sid * NPT + 12 * MEDG3, 128)], semZ)
        for z in range(12):
            pltpu.make_async_copy(
                rows.at[0], sp_agg.at[pl.ds(0, MEDG3)], semZ).wait()
        pltpu.make_async_copy(
            rows.at[0].at[pl.ds(0, 128)], sp_agg.at[pl.ds(0, 128)],
            semZ).wait()
        plsc.subcore_barrier()

        fire_l(0, 0)
        fire_d(0, 0)
        wait_l()
        addk(0, koff)
        fire_g(0, 0)
        fire_l(1, 1)
        drain_g(0)
        scale(0, 0)
        wait_d()
        fire_s(0, 0)
        fire_d(1, 1)
        wait_l()
        addk(1, koff)
        fire_g(1, 1)
        fire_l(2, 0)
        drain_g(1)
        scale(1, 1)
        wait_d()
        fire_s(1, 1)
        fire_d(2, 2)
        wait_l()
        addk(0, koff)
        fire_g(0, 2)
        fire_l(3, 1)

        def six(t, _):
            for u in range(6):
                m = 2 + u
                lp = m % 2
                lpn = (m + 1) % 2
                rp = m % 3
                rpn = (m + 1) % 3
                dp = m % 3
                dpn = (m + 1) % 3
                mm = t * 6 + 2 + u
                drain_g(rp)
                wait_l()
                addk(lpn, koff)
                drain_s()
                fire_g(lpn, rpn)
                scale(rp, lp)
                wait_d()
                fire_s(rp, dp)
                base = jnp.minimum(row0 + (mm + 2) * MR3,
                                   jnp.int32(E2 - MR3))
                pltpu.async_copy(src2d.at[pl.ds(base, MR3)],
                                 sidx.at[lp], semL)
                pltpu.async_copy(norm2d.at[pl.ds(base, MR3)],
                                 nrm.at[lp], semL)
                basd = jnp.minimum(row0 + (mm + 1) * MR3,
                                   jnp.int32(E2 - MR3))
                pltpu.async_copy(dst2d.at[pl.ds(basd, MR3)],
                                 didx.at[dpn], semD)
            return 0
        lax.fori_loop(0, (NM3 - 2) // 6, six, 0)

        wait_l()
        wait_d()
        drain_g(98 % 3)
        drain_s()
        drain_s()
        plsc.subcore_barrier()
        pltpu.sync_copy(sp_agg.at[pl.ds(sid * NPT, NPT)],
                        parts.at[cid, k, pl.ds(sid * NPT, NPT)])
        plsc.subcore_barrier()


MR4 = 8
NM4 = (EPT // 128) // MR4


@functools.partial(
    pl.kernel,
    out_type=jax.ShapeDtypeStruct((NC, K * N_PAD), F32),
    mesh=_sc_mesh,
    compiler_params=pltpu.CompilerParams(use_tc_tiling_on_sc=False),
    scratch_types=[
        pltpu.VMEM_SHARED((K * N_PAD,), F32),
        pltpu.VMEM_SHARED((K * N_PAD,), F32),
        pltpu.VMEM((2, MR4, 128), I32),
        pltpu.VMEM((2, MR4, 128), I32),
        pltpu.VMEM((2, MR4, 128), F32),
        pltpu.VMEM((3, MR4, 128), I32),
        pltpu.VMEM((3, MR4, 128), I32),
        pltpu.VMEM((3, MR4, 128), F32),
        pltpu.VMEM((NPT,), F32),
        pltpu.VMEM((NPT,), F32),
        pltpu.VMEM((NPT,), F32),
        pltpu.VMEM((L,), F32),
        pltpu.SemaphoreType.DMA,
        pltpu.SemaphoreType.DMA,
        pltpu.SemaphoreType.DMA,
    ],
)
def _s4_msg(pprev, cvec, w2row, src2d, dst2d, norm2d, parts,
            sp_h2, sp_agg, sidx, didx, nrm, gidx, didx2, vals,
            p0b, p1b, cb, wb, semL, semG, semS):
    cid = lax.axis_index("c")
    sid = lax.axis_index("s")
    wid = _wid()
    row0 = wid * (EPT // 128)

    def zb(i, _):
        p0b[pl.ds(i * L, L)] = jnp.zeros((L,), F32)
        return 0
    for k in range(K):
        off = k * N_PAD + sid * NPT
        pltpu.sync_copy(w2row.at[pl.ds(k * L, L)], wb)
        d1 = pltpu.async_copy(pprev.at[0, pl.ds(off, NPT)], p0b, semL)
        d2 = pltpu.async_copy(pprev.at[1, pl.ds(off, NPT)], p1b, semL)
        d3 = pltpu.async_copy(cvec.at[pl.ds(off, NPT)], cb, semL)
        d1.wait()
        d2.wait()
        d3.wait()
        wv = wb[pl.ds(0, L)]

        def mk(i, _):
            s = pl.ds(i * L, L)
            cb[s] = wv * (p0b[s] + p1b[s]) + cb[s]
            return 0
        lax.fori_loop(0, NPT // L, mk, 0)
        pltpu.sync_copy(cb, sp_h2.at[pl.ds(off, NPT)])
        lax.fori_loop(0, NPT // L, zb, 0)
        pltpu.sync_copy(p0b, sp_agg.at[pl.ds(off, NPT)])
    plsc.subcore_barrier()

    def fire_l(m, lp):
        base = jnp.minimum(row0 + m * MR4, jnp.int32(E2 - MR4))
        pltpu.async_copy(src2d.at[pl.ds(base, MR4)], sidx.at[lp], semL)
        pltpu.async_copy(dst2d.at[pl.ds(base, MR4)], didx.at[lp], semL)
        pltpu.async_copy(norm2d.at[pl.ds(base, MR4)], nrm.at[lp], semL)

    def wait_l():
        for _ in range(3):
            pltpu.make_async_copy(
                src2d.at[pl.ds(0, MR4)], sidx.at[0], semL).wait()

    def addk(lp, k):
        koff = jnp.int32(k * N_PAD)

        def body(r, _):
            for c in range(8):
                s = pl.ds(c * L, L)
                gidx[k, r, s] = sidx[lp, r, s] + koff
            return 0
        lax.fori_loop(0, MR4, body, 0)

    def fire_g(k):
        for j in range(MR4):
            pltpu.async_copy(sp_h2.at[gidx.at[k].at[j]], vals.at[k].at[j],
                             semG)

    def drain_g(k):
        for j in range(MR4):
            pltpu.make_async_copy(sp_h2.at[gidx.at[k].at[j]],
                                  vals.at[k].at[j], semG).wait()

    def mul(lp, k):
        koff = jnp.int32(k * N_PAD)

        def body(r, _):
            for c in range(8):
                s = pl.ds(c * L, L)
                vals[k, r, s] = vals[k, r, s] * nrm[lp, r, s]
                didx2[k, r, s] = didx[lp, r, s] + koff
            return 0
        lax.fori_loop(0, MR4, body, 0)

    def fire_s(k):
        for j in range(MR4):
            pltpu.async_copy(vals.at[k].at[j], sp_agg.at[didx2.at[k].at[j]],
                             semS, add=True)

    def drain_s(k):
        for j in range(MR4):
            pltpu.make_async_copy(vals.at[k].at[j],
                                  sp_agg.at[didx2.at[k].at[j]], semS).wait()

    fire_l(0, 0)
    wait_l()
    fire_l(1, 1)
    for k in range(K):
        addk(0, k)
        fire_g(k)
    for k in range(K):
        drain_g(k)
        mul(0, k)
        fire_s(k)

    def pair(t, _):
        for u in range(2):
            lp = (1 + u) % 2
            m = t * 2 + 1 + u
            wait_l()
            fire_l_base = jnp.minimum(row0 + (m + 1) * MR4,
                                      jnp.int32(E2 - MR4))
            pltpu.async_copy(src2d.at[pl.ds(fire_l_base, MR4)],
                             sidx.at[(m + 1) % 2], semL)
            pltpu.async_copy(dst2d.at[pl.ds(fire_l_base, MR4)],
                             didx.at[(m + 1) % 2], semL)
            pltpu.async_copy(norm2d.at[pl.ds(fire_l_base, MR4)],
                             nrm.at[(m + 1) % 2], semL)
            for k in range(K):
                addk(lp, k)
                drain_s(k)
                fire_g(k)
            for k in range(K):
                drain_g(k)
                mul(lp, k)
                fire_s(k)
        return 0
    lax.fori_loop(0, (NM4 - 1) // 2, pair, 0)

    wait_l()
    for k in range(K):
        drain_s(k)
    plsc.subcore_barrier()
    for k in range(K):
        off = k * N_PAD + sid * NPT
        pltpu.sync_copy(sp_agg.at[pl.ds(off, NPT)],
                        parts.at[cid, pl.ds(off, NPT)])


NP8 = N_PAD // 8
_T1B = 512
_NBLK1 = N_PAD // _T1B
_T2B = 1792
_NBLK2 = NP8 // _T2B


def _t1_body(xg_ref, rw_ref, iw_ref, b_ref, deg_ref, root_ref, h0_ref,
             dis_ref):
    xb = xg_ref[...]
    for k in range(K):
        root_ref[k] = jnp.dot(xb, rw_ref[k], preferred_element_type=F32) \
            + b_ref[k]
        h0_ref[k] = jnp.dot(xb, iw_ref[k], preferred_element_type=F32)
    d = deg_ref[0] + deg_ref[1]
    dis_ref[...] = jnp.where(d > 0.0, lax.rsqrt(jnp.abs(d) + 1e-30), 0.0)


_T1R = 1792


def _t1_call(xg, rwb, iwb, bb, deg2d):
    return pl.pallas_call(
        _t1_body,
        grid=(NP8 // _T1R,),
        in_specs=[
            pl.BlockSpec((_T1R, 8 * F_IN), lambda i: (i, 0)),
            pl.BlockSpec((K, 8 * F_IN, 128), lambda i: (0, 0, 0)),
            pl.BlockSpec((K, 8 * F_IN, 128), lambda i: (0, 0, 0)),
            pl.BlockSpec((K, 1, 128), lambda i: (0, 0, 0)),
            pl.BlockSpec((NC, _T1R // 16, 128), lambda i: (0, i, 0)),
        ],
        out_specs=[
            pl.BlockSpec((K, _T1R, 128), lambda i: (0, i, 0)),
            pl.BlockSpec((K, _T1R, 128), lambda i: (0, i, 0)),
            pl.BlockSpec((_T1R // 16, 128), lambda i: (i, 0)),
        ],
        out_shape=[
            jax.ShapeDtypeStruct((K, NP8, 128), F32),
            jax.ShapeDtypeStruct((K, NP8, 128), F32),
            jax.ShapeDtypeStruct((N_PAD // 128, 128), F32),
        ],
    )(xg, rwb, iwb, bb, deg2d)


def _t2_body(p_ref, root_ref, w_ref, h_ref):
    for k in range(K):
        out = jnp.maximum(p_ref[0, k] + p_ref[1, k] + root_ref[k], 0.0)
        h_ref[k] = jnp.dot(out, w_ref[k], preferred_element_type=F32)


def _t2_call(parts, rootb, w128):
    return pl.pallas_call(
        _t2_body,
        grid=(_NBLK2,),
        in_specs=[
            pl.BlockSpec((NC, K, _T2B, 128), lambda i: (0, 0, i, 0)),
            pl.BlockSpec((K, _T2B, 128), lambda i: (0, i, 0)),
            pl.BlockSpec((K, 128, 128), lambda i: (0, 0, 0)),
        ],
        out_specs=pl.BlockSpec((K, _T2B, 128), lambda i: (0, i, 0)),
        out_shape=jax.ShapeDtypeStruct((K, NP8, 128), F32),
    )(parts, rootb, w128)


def _t2b_body(p_ref, root_ref, sc_ref, sh_ref, rw2_ref, iw2_ref, b2_ref,
              w2_ref, root2_ref, h20_ref, c1_ref):
    acc = jnp.zeros((_T2B, 128), F32)
    for k in range(K):
        acc = acc + jnp.maximum(p_ref[0, k] + p_ref[1, k] + root_ref[k], 0.0)
    hm = acc * (1.0 / K)
    hbn = jnp.maximum(hm * sc_ref[...] + sh_ref[...], 0.0)
    for k in range(K):
        r2 = jnp.dot(hbn, rw2_ref[k], preferred_element_type=F32) \
            + b2_ref[k, 0, 0]
        h2 = jnp.dot(hbn, iw2_ref[k], preferred_element_type=F32)
        root2_ref[k] = r2
        h20_ref[k] = h2
        c1_ref[k] = r2 * w2_ref[k, 0, 0]


def _t2b_call(parts, rootb, bn_scale, bn_shift, rw2b, iw2b, b2, w2):
    return pl.pallas_call(
        _t2b_body,
        grid=(_NBLK2,),
        in_specs=[
            pl.BlockSpec((NC, K, _T2B, 128), lambda i: (0, 0, i, 0)),
            pl.BlockSpec((K, _T2B, 128), lambda i: (0, i, 0)),
            pl.BlockSpec((1, 128), lambda i: (0, 0)),
            pl.BlockSpec((1, 128), lambda i: (0, 0)),
            pl.BlockSpec((K, 128, 8), lambda i: (0, 0, 0)),
            pl.BlockSpec((K, 128, 8), lambda i: (0, 0, 0)),
            pl.BlockSpec((K, 1, 1), lambda i: (0, 0, 0)),
            pl.BlockSpec((K, 1, 1), lambda i: (0, 0, 0)),
        ],
        out_specs=[
            pl.BlockSpec((K, _T2B, 8), lambda i: (0, i, 0)),
            pl.BlockSpec((K, _T2B, 8), lambda i: (0, i, 0)),
            pl.BlockSpec((K, _T2B, 8), lambda i: (0, i, 0)),
        ],
        out_shape=[
            jax.ShapeDtypeStruct((K, NP8, 8), F32),
            jax.ShapeDtypeStruct((K, NP8, 8), F32),
            jax.ShapeDtypeStruct((K, NP8, 8), F32),
        ],
    )(parts, rootb, bn_scale, bn_shift, rw2b, iw2b, b2, w2)


NR128 = N_PAD // 128


def _f1_body(p_ref, root2_ref, o_ref):
    s = jnp.zeros((NR128, 128), F32)
    for k in range(K):
        s = s + p_ref[0, k] + p_ref[1, k] + root2_ref[k]
    o_ref[...] = jax.nn.sigmoid(s * (1.0 / K))


def _f1_call(parts, root2b):
    return pl.pallas_call(
        _f1_body,
        in_specs=[
            pl.BlockSpec((NC, K, NR128, 128), lambda: (0, 0, 0, 0)),
            pl.BlockSpec((K, NR128, 128), lambda: (0, 0, 0)),
        ],
        out_specs=pl.BlockSpec((NR128, 128), lambda: (0, 0)),
        out_shape=jax.ShapeDtypeStruct((NR128, 128), F32),
    )(parts, root2b)


def kernel(x, edge_index, edge_attr, batch,
           conv1_init_w, conv1_w, conv1_root_w, conv1_bias,
           bn1_gamma, bn1_beta, bn1_mean, bn1_var,
           conv2_init_w, conv2_w, conv2_root_w, conv2_bias):
    del batch
    pad = E_PAD - E
    fill = (jnp.arange(pad, dtype=I32) * 37) % N
    src = jnp.concatenate([edge_index[0].astype(I32), fill]).reshape(E2, 128)
    dst = jnp.concatenate([edge_index[1].astype(I32), fill]).reshape(E2, 128)
    ea = jnp.concatenate([edge_attr.astype(F32),
                          jnp.zeros((pad,), F32)]).reshape(E2, 128)

    eye8 = jnp.eye(8, dtype=F32)
    xg = x.reshape(NP8 // 196 * 196, 8 * F_IN) if False else \
        jnp.pad(x, ((0, N_PAD - N), (0, 0))).reshape(NP8, 8 * F_IN)
    rwb1 = jnp.einsum("ab,kij->kaibj", eye8,
                      conv1_root_w).reshape(K, 8 * F_IN, 128)
    iwb1 = jnp.einsum("ab,kij->kaibj", eye8,
                      conv1_init_w).reshape(K, 8 * F_IN, 128)
    bb1 = jnp.tile(conv1_bias, (1, 1, 8)).reshape(K, 1, 128)

    deg_parts = _s1_deg(dst, ea)
    rootb, h0, dis = _t1_call(xg, rwb1, iwb1, bb1,
                              deg_parts.reshape(NC, N_PAD // 128, 128))
    norm2d = _s2_norm(dis.reshape(N_PAD), src, dst, ea)

    w128 = jnp.einsum("ab,kij->kaibj", eye8, conv1_w).reshape(K, 128, 128)
    rw2b = jnp.einsum("ab,kij->kaibj", eye8, conv2_root_w).reshape(K, 128, 8)
    iw2b = jnp.einsum("ab,kij->kaibj", eye8, conv2_init_w).reshape(K, 128, 8)
    bn_scale16 = bn1_gamma * lax.rsqrt(bn1_var + 1e-5)
    bn_shift16 = bn1_beta - bn1_mean * bn_scale16
    bn_scale = jnp.tile(bn_scale16, 8).reshape(1, 128)
    bn_shift = jnp.tile(bn_shift16, 8).reshape(1, 128)

    rootb_p = rootb
    h = h0.reshape(K * N_PAD, HID)
    for _ in range(T - 1):
        parts1 = _s3_msg(h, src, dst, norm2d)
        h = _t2_call(parts1.reshape(NC, K, NP8, 128), rootb_p,
                     w128).reshape(K * N_PAD, HID)
    parts1 = _s3_msg(h, src, dst, norm2d)
    root2b, h20, c1 = _t2b_call(
        parts1.reshape(NC, K, NP8, 128), rootb_p, bn_scale, bn_shift,
        rw2b, iw2b, conv2_bias, conv2_w)

    w2row = jnp.broadcast_to(
        conv2_w.reshape(K, 1).astype(F32), (K, L)).reshape(K * L)
    zeros2 = jnp.zeros((NC, K * N_PAD), F32)
    cvec = h20.reshape(K * N_PAD)
    c1f = c1.reshape(K * N_PAD)
    parts2 = _s4_msg(zeros2, cvec, w2row, src, dst, norm2d)
    for _ in range(T - 1):
        parts2 = _s4_msg(parts2, c1f, w2row, src, dst, norm2d)

    out = _f1_call(parts2.reshape(NC, K, NR128, 128),
                   root2b.reshape(K, NR128, 128))
    return out.reshape(N_PAD)[:N].reshape(N, 1)
